# baseline (device time: 370206 ns/iter reference)
import jax
import jax.numpy as jnp
from jax import lax
from jax.experimental import pallas as pl
from jax.experimental.pallas import tpu as pltpu

N_DEV = 4
M_PER = 2048
K = 8192
K_PER = 2048
N = 4096

KT = 128
P = K_PER // KT
D_ORDER = (1, 3, 2)
N_REMOTE = len(D_ORDER) * P


def _fused_body(x_ref, w_ref, y_ref, a_ref,
                y_acc, a_vmem, w_vmem,
                a_sems, w_sems, send_sems, recv_sems):
    my = lax.axis_index("i")

    barrier = pltpu.get_barrier_semaphore()
    for d in range(1, N_DEV):
        peer = lax.rem(my + d, N_DEV)
        pl.semaphore_signal(
            barrier, inc=1, device_id=(peer,),
            device_id_type=pl.DeviceIdType.MESH,
        )
    pl.semaphore_wait(barrier, N_DEV - 1)

    sends = []
    for kc in range(P):
        for d in D_ORDER:
            t = lax.rem(my + d, N_DEV)
            rdma = pltpu.make_async_remote_copy(
                src_ref=x_ref.at[pl.ds(t * M_PER, M_PER),
                                 pl.ds(kc * KT, KT)],
                dst_ref=a_ref.at[:, pl.ds(my * K_PER + kc * KT, KT)],
                send_sem=send_sems.at[d - 1, kc],
                recv_sem=recv_sems.at[d - 1, kc],
                device_id=(t,),
                device_id_type=pl.DeviceIdType.MESH,
            )
            rdma.start()
            sends.append(rdma)

    y_acc[...] = jnp.zeros_like(y_acc)


    def wait_piece(d, kc):
        src_dev = lax.rem(my - d + N_DEV, N_DEV)
        dst = a_ref.at[:, pl.ds(src_dev * K_PER + kc * KT, KT)]
        rdma = pltpu.make_async_remote_copy(
            src_ref=dst, dst_ref=dst,
            send_sem=recv_sems.at[d - 1, kc],
            recv_sem=recv_sems.at[d - 1, kc],
            device_id=(my,),
            device_id_type=pl.DeviceIdType.MESH,
        )
        rdma.wait_recv()

    def start_w(kblock, slot):
        cp = pltpu.make_async_copy(
            w_ref.at[pl.ds(kblock * KT, KT), :], w_vmem.at[slot],
            w_sems.at[slot])
        cp.start()

    def start_a_local(kc, slot):
        cp = pltpu.make_async_copy(
            x_ref.at[pl.ds(my * M_PER, M_PER), pl.ds(kc * KT, KT)],
            a_vmem.at[slot], a_sems.at[slot])
        cp.start()

    def start_a_remote(d, kc, slot):
        src_dev = lax.rem(my - d + N_DEV, N_DEV)
        cp = pltpu.make_async_copy(
            a_ref.at[:, pl.ds(src_dev * K_PER + kc * KT, KT)],
            a_vmem.at[slot], a_sems.at[slot])
        cp.start()

    def wait_ab(slot):
        pltpu.make_async_copy(
            a_vmem.at[slot], a_vmem.at[slot], a_sems.at[slot]).wait()
        pltpu.make_async_copy(
            w_vmem.at[slot], w_vmem.at[slot], w_sems.at[slot]).wait()

    def compute(slot):
        y_acc[...] += jnp.dot(
            a_vmem[slot].astype(jnp.float32), w_vmem[slot],
            preferred_element_type=jnp.float32)

    def rem_step(j):
        r = lax.rem(j, 3)
        d = jnp.where(r == 0, D_ORDER[0],
                      jnp.where(r == 1, D_ORDER[1], D_ORDER[2]))
        kc = lax.div(j, 3)
        return d, kc

    start_w(my * P + 0, 0)
    start_a_local(0, 0)

    def local_body(kc, _):
        slot = lax.rem(kc, 2)
        nslot = lax.rem(kc + 1, 2)

        @pl.when(kc + 1 < P)
        def _():
            start_w(my * P + kc + 1, nslot)
            start_a_local(kc + 1, nslot)

        @pl.when(kc + 1 == P)
        def _():
            d, kc0 = rem_step(0)
            wait_piece(d, kc0)
            src_dev = lax.rem(my - d + N_DEV, N_DEV)
            start_w(src_dev * P + kc0, nslot)
            start_a_remote(d, kc0, nslot)

        wait_ab(slot)
        compute(slot)
        return 0

    lax.fori_loop(0, P, local_body, 0)

    def remote_body(j, _):
        slot = lax.rem(P + j, 2)
        nslot = lax.rem(P + j + 1, 2)

        @pl.when(j + 1 < N_REMOTE)
        def _():
            d, kc = rem_step(j + 1)
            wait_piece(d, kc)
            src_dev = lax.rem(my - d + N_DEV, N_DEV)
            start_w(src_dev * P + kc, nslot)
            start_a_remote(d, kc, nslot)

        wait_ab(slot)
        compute(slot)
        return 0

    lax.fori_loop(0, N_REMOTE, remote_body, 0)

    out_cp = pltpu.make_async_copy(y_acc, y_ref, a_sems.at[0])
    out_cp.start()
    out_cp.wait()

    for rdma in sends:
        rdma.wait_send()


_CAST_ROWS = 1024


def _cast_body(x_ref, o_ref):
    o_ref[...] = x_ref[...].astype(jnp.bfloat16)


def _cast_bf16(x):
    m, k = x.shape
    return pl.pallas_call(
        _cast_body,
        grid=(m // _CAST_ROWS,),
        in_specs=[pl.BlockSpec((_CAST_ROWS, k), lambda i: (i, 0))],
        out_specs=pl.BlockSpec((_CAST_ROWS, k), lambda i: (i, 0)),
        out_shape=jax.ShapeDtypeStruct((m, k), jnp.bfloat16),
    )(x)


def kernel(x, w_mat):
    x = _cast_bf16(x)
    wire = x.dtype
    y, _ = pl.pallas_call(
        _fused_body,
        out_shape=[
            jax.ShapeDtypeStruct((M_PER, N), jnp.float32),
            jax.ShapeDtypeStruct((M_PER, K), wire),
        ],
        in_specs=[
            pl.BlockSpec(memory_space=pl.ANY),
            pl.BlockSpec(memory_space=pl.ANY),
        ],
        out_specs=[
            pl.BlockSpec(memory_space=pl.ANY),
            pl.BlockSpec(memory_space=pl.ANY),
        ],
        scratch_shapes=[
            pltpu.VMEM((M_PER, N), jnp.float32),
            pltpu.VMEM((2, M_PER, KT), wire),
            pltpu.VMEM((2, KT, N), jnp.float32),
            pltpu.SemaphoreType.DMA((2,)),
            pltpu.SemaphoreType.DMA((2,)),
            pltpu.SemaphoreType.DMA((N_DEV - 1, P)),
            pltpu.SemaphoreType.DMA((N_DEV - 1, P)),
        ],
        compiler_params=pltpu.CompilerParams(
            collective_id=0,
            vmem_limit_bytes=63 * 1024 * 1024,
        ),
    )(x, w_mat)
    return y


# device time: 261491 ns/iter; 1.4158x vs baseline; 1.4158x over previous
import jax
import jax.numpy as jnp
from jax import lax
from jax.experimental import pallas as pl
from jax.experimental.pallas import tpu as pltpu

N_DEV = 4
M_PER = 2048
K = 8192
K_PER = 2048
N = 4096

KT = 256
P = K_PER // KT
D_ORDER = (1, 3, 2)
N_REMOTE = len(D_ORDER) * P


def _fused_body(x_ref, w_ref, y_ref, a_ref,
                y_acc, a_vmem, w_vmem,
                a_sems, w_sems, send_sems, recv_sems):
    my = lax.axis_index("i")

    barrier = pltpu.get_barrier_semaphore()
    for d in range(1, N_DEV):
        peer = lax.rem(my + d, N_DEV)
        pl.semaphore_signal(
            barrier, inc=1, device_id=(peer,),
            device_id_type=pl.DeviceIdType.MESH,
        )
    pl.semaphore_wait(barrier, N_DEV - 1)

    sends = []
    for kc in range(P):
        for d in D_ORDER:
            t = lax.rem(my + d, N_DEV)
            rdma = pltpu.make_async_remote_copy(
                src_ref=x_ref.at[pl.ds(t * M_PER, M_PER),
                                 pl.ds(kc * KT, KT)],
                dst_ref=a_ref.at[:, pl.ds(my * K_PER + kc * KT, KT)],
                send_sem=send_sems.at[d - 1, kc],
                recv_sem=recv_sems.at[d - 1, kc],
                device_id=(t,),
                device_id_type=pl.DeviceIdType.MESH,
            )
            rdma.start()
            sends.append(rdma)

    y_acc[...] = jnp.zeros_like(y_acc)


    def wait_piece(d, kc):
        src_dev = lax.rem(my - d + N_DEV, N_DEV)
        dst = a_ref.at[:, pl.ds(src_dev * K_PER + kc * KT, KT)]
        rdma = pltpu.make_async_remote_copy(
            src_ref=dst, dst_ref=dst,
            send_sem=recv_sems.at[d - 1, kc],
            recv_sem=recv_sems.at[d - 1, kc],
            device_id=(my,),
            device_id_type=pl.DeviceIdType.MESH,
        )
        rdma.wait_recv()

    def start_w(kblock, slot):
        cp = pltpu.make_async_copy(
            w_ref.at[pl.ds(kblock * KT, KT), :], w_vmem.at[slot],
            w_sems.at[slot])
        cp.start()

    def start_a_local(kc, slot):
        cp = pltpu.make_async_copy(
            x_ref.at[pl.ds(my * M_PER, M_PER), pl.ds(kc * KT, KT)],
            a_vmem.at[slot], a_sems.at[slot])
        cp.start()

    def start_a_remote(d, kc, slot):
        src_dev = lax.rem(my - d + N_DEV, N_DEV)
        cp = pltpu.make_async_copy(
            a_ref.at[:, pl.ds(src_dev * K_PER + kc * KT, KT)],
            a_vmem.at[slot], a_sems.at[slot])
        cp.start()

    def wait_ab(slot):
        pltpu.make_async_copy(
            a_vmem.at[slot], a_vmem.at[slot], a_sems.at[slot]).wait()
        pltpu.make_async_copy(
            w_vmem.at[slot], w_vmem.at[slot], w_sems.at[slot]).wait()

    def compute(slot):
        y_acc[...] += jnp.dot(
            a_vmem[slot].astype(jnp.float32), w_vmem[slot],
            preferred_element_type=jnp.float32)

    def rem_step(j):
        r = lax.rem(j, 3)
        d = jnp.where(r == 0, D_ORDER[0],
                      jnp.where(r == 1, D_ORDER[1], D_ORDER[2]))
        kc = lax.div(j, 3)
        return d, kc

    start_w(my * P + 0, 0)
    start_a_local(0, 0)

    def local_body(kc, _):
        slot = lax.rem(kc, 2)
        nslot = lax.rem(kc + 1, 2)

        @pl.when(kc + 1 < P)
        def _():
            start_w(my * P + kc + 1, nslot)
            start_a_local(kc + 1, nslot)

        @pl.when(kc + 1 == P)
        def _():
            d, kc0 = rem_step(0)
            wait_piece(d, kc0)
            src_dev = lax.rem(my - d + N_DEV, N_DEV)
            start_w(src_dev * P + kc0, nslot)
            start_a_remote(d, kc0, nslot)

        wait_ab(slot)
        compute(slot)
        return 0

    lax.fori_loop(0, P, local_body, 0)

    def remote_body(j, _):
        slot = lax.rem(P + j, 2)
        nslot = lax.rem(P + j + 1, 2)

        @pl.when(j + 1 < N_REMOTE)
        def _():
            d, kc = rem_step(j + 1)
            wait_piece(d, kc)
            src_dev = lax.rem(my - d + N_DEV, N_DEV)
            start_w(src_dev * P + kc, nslot)
            start_a_remote(d, kc, nslot)

        wait_ab(slot)
        compute(slot)
        return 0

    lax.fori_loop(0, N_REMOTE, remote_body, 0)

    out_cp = pltpu.make_async_copy(y_acc, y_ref, a_sems.at[0])
    out_cp.start()
    out_cp.wait()

    for rdma in sends:
        rdma.wait_send()


_CAST_ROWS = 1024


def _cast_body(x_ref, o_ref):
    o_ref[...] = x_ref[...].astype(jnp.bfloat16)


def _cast_bf16(x):
    m, k = x.shape
    return pl.pallas_call(
        _cast_body,
        grid=(m // _CAST_ROWS,),
        in_specs=[pl.BlockSpec((_CAST_ROWS, k), lambda i: (i, 0))],
        out_specs=pl.BlockSpec((_CAST_ROWS, k), lambda i: (i, 0)),
        out_shape=jax.ShapeDtypeStruct((m, k), jnp.bfloat16),
    )(x)


def kernel(x, w_mat):
    x = _cast_bf16(x)
    wire = x.dtype
    y, _ = pl.pallas_call(
        _fused_body,
        out_shape=[
            jax.ShapeDtypeStruct((M_PER, N), jnp.float32),
            jax.ShapeDtypeStruct((M_PER, K), wire),
        ],
        in_specs=[
            pl.BlockSpec(memory_space=pl.ANY),
            pl.BlockSpec(memory_space=pl.ANY),
        ],
        out_specs=[
            pl.BlockSpec(memory_space=pl.ANY),
            pl.BlockSpec(memory_space=pl.ANY),
        ],
        scratch_shapes=[
            pltpu.VMEM((M_PER, N), jnp.float32),
            pltpu.VMEM((2, M_PER, KT), wire),
            pltpu.VMEM((2, KT, N), jnp.float32),
            pltpu.SemaphoreType.DMA((2,)),
            pltpu.SemaphoreType.DMA((2,)),
            pltpu.SemaphoreType.DMA((N_DEV - 1, P)),
            pltpu.SemaphoreType.DMA((N_DEV - 1, P)),
        ],
        compiler_params=pltpu.CompilerParams(
            collective_id=0,
            vmem_limit_bytes=63 * 1024 * 1024,
        ),
    )(x, w_mat)
    return y
